# Initial kernel scaffold; baseline (speedup 1.0000x reference)
import jax
import jax.numpy as jnp
from jax.experimental import pallas as pl


def kernel(v, a_int, a_nh, W1i, A1i, W1n, A1n, W2i, A2i, W2n, A2n, W3i, A3i, W3n, A3n, Wd, bd):
    raise NotImplementedError("write your pallas kernel here")


# speedup vs baseline: 1.1806x; 1.1806x over previous
import jax
import jax.numpy as jnp
from jax.experimental import pallas as pl
from jax.experimental.pallas import tpu as pltpu

N = 2048
H = 6
BR = 256
NR = N // BR


def _mask_prep(a):
    def body(a_ref, m_ref):
        i = pl.program_id(0)
        rows = jax.lax.broadcasted_iota(jnp.int32, (BR, N), 0) + i * BR
        cols = jax.lax.broadcasted_iota(jnp.int32, (BR, N), 1)
        m = (a_ref[...] > 0.99) | (rows == cols)
        m_ref[...] = m.astype(jnp.int8)

    return pl.pallas_call(
        body,
        grid=(NR,),
        in_specs=[pl.BlockSpec((BR, N), lambda i: (i, 0))],
        out_specs=pl.BlockSpec((BR, N), lambda i: (i, 0)),
        out_shape=jax.ShapeDtypeStruct((N, N), jnp.int8),
    )(a)


def _gat(xh, mask8, Wr, attL, attR):
    Hp, _, Fp = xh.shape
    Fo = Wr.shape[-1]

    def body(x_ref, w_ref, al_ref, ar_ref, m_ref, o_ref, h_scr, fl_scr, fr_scr):
        i = pl.program_id(1)

        @pl.when(i == 0)
        def _project():
            acc = jnp.zeros((N, Fo), jnp.float32)
            for g in range(Hp):
                acc = acc + jnp.dot(x_ref[g], w_ref[0, g],
                                    preferred_element_type=jnp.float32)
            h_scr[...] = acc
            fl_scr[...] = jax.lax.dot_general(
                acc, al_ref[0], (((1,), (1,)), ((), ())),
                preferred_element_type=jnp.float32)
            fr_scr[...] = jax.lax.dot_general(
                ar_ref[0], acc, (((1,), (1,)), ((), ())),
                preferred_element_type=jnp.float32)

        fl = fl_scr[pl.ds(i * BR, BR), :]
        e = fl + fr_scr[...]
        e = jnp.where(e > 0, e, 0.2 * e)
        m = m_ref[...] != 0
        em = jnp.where(m, e, jnp.float32(-1e9))
        rm = jnp.max(em, axis=1, keepdims=True)
        p = jnp.where(m, jnp.exp(e - rm), 0.0)
        s = jnp.sum(p, axis=1, keepdims=True)
        o = jnp.dot(p, h_scr[...], preferred_element_type=jnp.float32) / s
        o_ref[0] = jnp.maximum(o, 0.0)

    return pl.pallas_call(
        body,
        grid=(H, NR),
        in_specs=[
            pl.BlockSpec((Hp, N, Fp), lambda h, i: (0, 0, 0)),
            pl.BlockSpec((1, Hp, Fp, Fo), lambda h, i: (h, 0, 0, 0)),
            pl.BlockSpec((1, 1, Fo), lambda h, i: (h, 0, 0)),
            pl.BlockSpec((1, 1, Fo), lambda h, i: (h, 0, 0)),
            pl.BlockSpec((BR, N), lambda h, i: (i, 0)),
        ],
        out_specs=pl.BlockSpec((1, BR, Fo), lambda h, i: (h, i, 0)),
        out_shape=jax.ShapeDtypeStruct((H, N, Fo), jnp.float32),
        scratch_shapes=[
            pltpu.VMEM((N, Fo), jnp.float32),
            pltpu.VMEM((N, 1), jnp.float32),
            pltpu.VMEM((1, N), jnp.float32),
        ],
    )(xh, Wr, attL, attR, mask8)


def _readout(hi, hn, wdi, wdn, bd):
    def body(hi_ref, hn_ref, wdi_ref, wdn_ref, bd_ref, o_ref):
        si = jnp.sum(hi_ref[...], axis=1)
        sn = jnp.sum(hn_ref[...], axis=1)
        def tot(x):
            return jnp.sum(x, axis=(0, 1), keepdims=True)
        n2 = tot(si * si) + tot(sn * sn)
        nrm = jnp.maximum(jnp.sqrt(n2), jnp.float32(1e-12))
        y = (tot(si * wdi_ref[...]) + tot(sn * wdn_ref[...])) / nrm
        o_ref[...] = y + bd_ref[...]

    return pl.pallas_call(
        body,
        out_shape=jax.ShapeDtypeStruct((1, 1), jnp.float32),
    )(hi, hn, wdi, wdn, bd)


def kernel(v, a_int, a_nh, W1i, A1i, W1n, A1n, W2i, A2i, W2n, A2n,
           W3i, A3i, W3n, A3n, Wd, bd):
    m_int = _mask_prep(a_int)
    m_nh = _mask_prep(a_nh)

    def split_att(A, Fo):
        return A[:, None, :Fo], A[:, None, Fo:]

    x0 = v.reshape(1, N, 11)
    a1l_i, a1r_i = split_att(A1i, 16)
    a1l_n, a1r_n = split_att(A1n, 16)
    hi = _gat(x0, m_int, W1i.reshape(H, 1, 11, 16), a1l_i, a1r_i)
    hn = _gat(x0, m_nh, W1n.reshape(H, 1, 11, 16), a1l_n, a1r_n)

    a2l_i, a2r_i = split_att(A2i, 32)
    a2l_n, a2r_n = split_att(A2n, 32)
    hi = _gat(hi, m_int, W2i.reshape(H, H, 16, 32), a2l_i, a2r_i)
    hn = _gat(hn, m_nh, W2n.reshape(H, H, 16, 32), a2l_n, a2r_n)

    a3l_i, a3r_i = split_att(A3i, 64)
    a3l_n, a3r_n = split_att(A3n, 64)
    hi = _gat(hi, m_int, W3i.reshape(H, H, 32, 64), a3l_i, a3r_i)
    hn = _gat(hn, m_nh, W3n.reshape(H, H, 32, 64), a3l_n, a3r_n)

    wdi = Wd[:H * 64, 0].reshape(H, 64)
    wdn = Wd[H * 64:, 0].reshape(H, 64)
    out = _readout(hi, hn, wdi, wdn, bd.reshape(1, 1))
    return out.reshape(1)


# speedup vs baseline: 1.4751x; 1.2495x over previous
import jax
import jax.numpy as jnp
from jax.experimental import pallas as pl
from jax.experimental.pallas import tpu as pltpu

N = 2048
H = 6
BR = 256
NR = N // BR


def _mask_prep(a):
    def body(a_ref, m_ref):
        i = pl.program_id(0)
        rows = jax.lax.broadcasted_iota(jnp.int32, (BR, N), 0) + i * BR
        cols = jax.lax.broadcasted_iota(jnp.int32, (BR, N), 1)
        m = (a_ref[...] > 0.99) | (rows == cols)
        m_ref[...] = m.astype(jnp.int8)

    return pl.pallas_call(
        body,
        grid=(NR,),
        in_specs=[pl.BlockSpec((BR, N), lambda i: (i, 0))],
        out_specs=pl.BlockSpec((BR, N), lambda i: (i, 0)),
        out_shape=jax.ShapeDtypeStruct((N, N), jnp.int8),
    )(a)


def _gat(xh, mask8, Wr, attL, attR):
    Hp, _, Fp = xh.shape
    Fo = Wr.shape[-1]

    def body(x_ref, w_ref, al_ref, ar_ref, m_ref, o_ref, h_scr, fl_scr, fr_scr):
        i = pl.program_id(1)

        @pl.when(i == 0)
        def _project():
            acc = jnp.zeros((N, Fo), jnp.float32)
            for g in range(Hp):
                acc = acc + jnp.dot(x_ref[g], w_ref[0, g],
                                    preferred_element_type=jnp.float32)
            h_scr[:, :Fo] = acc
            h_scr[:, Fo:] = jnp.ones((N, 1), jnp.float32)
            fl_scr[...] = jax.lax.dot_general(
                acc, al_ref[0], (((1,), (1,)), ((), ())),
                preferred_element_type=jnp.float32)
            fr_scr[...] = jax.lax.dot_general(
                ar_ref[0], acc, (((1,), (1,)), ((), ())),
                preferred_element_type=jnp.float32)

        fl = fl_scr[pl.ds(i * BR, BR), :]
        e = fl + fr_scr[...]
        e = jnp.maximum(e, 0.2 * e)
        em = jnp.where(m_ref[...] != 0, e, jnp.float32(-1e9))
        rm = jnp.max(em, axis=1, keepdims=True)
        p = jnp.exp2(em - rm)
        oext = jnp.dot(p, h_scr[...], preferred_element_type=jnp.float32)
        o = oext[:, :Fo] / oext[:, Fo:]
        o_ref[0] = jnp.maximum(o, 0.0)

    return pl.pallas_call(
        body,
        grid=(H, NR),
        in_specs=[
            pl.BlockSpec((Hp, N, Fp), lambda h, i: (0, 0, 0)),
            pl.BlockSpec((1, Hp, Fp, Fo), lambda h, i: (h, 0, 0, 0)),
            pl.BlockSpec((1, 1, Fo), lambda h, i: (h, 0, 0)),
            pl.BlockSpec((1, 1, Fo), lambda h, i: (h, 0, 0)),
            pl.BlockSpec((BR, N), lambda h, i: (i, 0)),
        ],
        out_specs=pl.BlockSpec((1, BR, Fo), lambda h, i: (h, i, 0)),
        out_shape=jax.ShapeDtypeStruct((H, N, Fo), jnp.float32),
        scratch_shapes=[
            pltpu.VMEM((N, Fo + 1), jnp.float32),
            pltpu.VMEM((N, 1), jnp.float32),
            pltpu.VMEM((1, N), jnp.float32),
        ],
    )(xh, Wr, attL, attR, mask8)


def _readout(hi, hn, wdi, wdn, bd):
    def body(hi_ref, hn_ref, wdi_ref, wdn_ref, bd_ref, o_ref):
        si = jnp.sum(hi_ref[...], axis=1)
        sn = jnp.sum(hn_ref[...], axis=1)
        def tot(x):
            return jnp.sum(x, axis=(0, 1), keepdims=True)
        n2 = tot(si * si) + tot(sn * sn)
        nrm = jnp.maximum(jnp.sqrt(n2), jnp.float32(1e-12))
        y = (tot(si * wdi_ref[...]) + tot(sn * wdn_ref[...])) / nrm
        o_ref[...] = y + bd_ref[...]

    return pl.pallas_call(
        body,
        out_shape=jax.ShapeDtypeStruct((1, 1), jnp.float32),
    )(hi, hn, wdi, wdn, bd)


def kernel(v, a_int, a_nh, W1i, A1i, W1n, A1n, W2i, A2i, W2n, A2n,
           W3i, A3i, W3n, A3n, Wd, bd):
    m_int = _mask_prep(a_int)
    m_nh = _mask_prep(a_nh)

    def split_att(A, Fo):
        log2e = jnp.float32(1.4426950408889634)
        return A[:, None, :Fo] * log2e, A[:, None, Fo:] * log2e

    x0 = v.reshape(1, N, 11)
    a1l_i, a1r_i = split_att(A1i, 16)
    a1l_n, a1r_n = split_att(A1n, 16)
    hi = _gat(x0, m_int, W1i.reshape(H, 1, 11, 16), a1l_i, a1r_i)
    hn = _gat(x0, m_nh, W1n.reshape(H, 1, 11, 16), a1l_n, a1r_n)

    a2l_i, a2r_i = split_att(A2i, 32)
    a2l_n, a2r_n = split_att(A2n, 32)
    hi = _gat(hi, m_int, W2i.reshape(H, H, 16, 32), a2l_i, a2r_i)
    hn = _gat(hn, m_nh, W2n.reshape(H, H, 16, 32), a2l_n, a2r_n)

    a3l_i, a3r_i = split_att(A3i, 64)
    a3l_n, a3r_n = split_att(A3n, 64)
    hi = _gat(hi, m_int, W3i.reshape(H, H, 32, 64), a3l_i, a3r_i)
    hn = _gat(hn, m_nh, W3n.reshape(H, H, 32, 64), a3l_n, a3r_n)

    wdi = Wd[:H * 64, 0].reshape(H, 64)
    wdn = Wd[H * 64:, 0].reshape(H, 64)
    out = _readout(hi, hn, wdi, wdn, bd.reshape(1, 1))
    return out.reshape(1)


# speedup vs baseline: 1.8467x; 1.2519x over previous
import jax
import jax.numpy as jnp
from jax.experimental import pallas as pl
from jax.experimental.pallas import tpu as pltpu

N = 2048
H = 6
BR = 256
NR = N // BR
MASK_OFF = -1080.0


def _mask_prep(a):
    def body(a_ref, m_ref):
        i = pl.program_id(0)
        rows = jax.lax.broadcasted_iota(jnp.int32, (BR, N), 0) + i * BR
        cols = jax.lax.broadcasted_iota(jnp.int32, (BR, N), 1)
        m = (a_ref[...] > 0.99) | (rows == cols)
        m_ref[...] = jnp.where(m, 0.0, MASK_OFF).astype(jnp.bfloat16)

    return pl.pallas_call(
        body,
        grid=(NR,),
        in_specs=[pl.BlockSpec((BR, N), lambda i: (i, 0))],
        out_specs=pl.BlockSpec((BR, N), lambda i: (i, 0)),
        out_shape=jax.ShapeDtypeStruct((N, N), jnp.bfloat16),
    )(a)


def _gat(xh, maskadd, Wr, attL, attR):
    Hp, _, Fp = xh.shape
    Fo = Wr.shape[-1]

    def body(x_ref, w_ref, al_ref, ar_ref, m_ref, o_ref,
             h_scr, fl_scr, fr_scr, d_scr):
        i = pl.program_id(0)
        hd = pl.program_id(1)

        @pl.when((i == 0) & (hd == 0))
        def _project():
            for g in range(H):
                acc = jnp.zeros((N, Fo), jnp.float32)
                for gp in range(Hp):
                    acc = acc + jnp.dot(x_ref[gp], w_ref[g, gp],
                                        preferred_element_type=jnp.float32)
                h_scr[g, :, :Fo] = acc
                h_scr[g, :, Fo:] = jnp.ones((N, 1), jnp.float32)
                fl = jax.lax.dot_general(
                    acc, al_ref[g], (((1,), (1,)), ((), ())),
                    preferred_element_type=jnp.float32)
                fr_row = jax.lax.dot_general(
                    ar_ref[g], acc, (((1,), (1,)), ((), ())),
                    preferred_element_type=jnp.float32)
                fr_col = jax.lax.dot_general(
                    acc, ar_ref[g], (((1,), (1,)), ((), ())),
                    preferred_element_type=jnp.float32)
                fl_scr[g] = fl
                fr_scr[g] = fr_row
                fd = fl + fr_col
                d_scr[g] = jnp.maximum(fd, 0.2 * fd)

        fl = fl_scr[hd, pl.ds(i * BR, BR), :]
        d = d_scr[hd, pl.ds(i * BR, BR), :]
        e = fl + fr_scr[hd]
        e = jnp.maximum(e, 0.2 * e)
        p = jnp.exp2(e + m_ref[...].astype(jnp.float32) - d)
        oext = jnp.dot(p, h_scr[hd], preferred_element_type=jnp.float32)
        o = oext[:, :Fo] / oext[:, Fo:]
        o_ref[0] = jnp.maximum(o, 0.0)

    return pl.pallas_call(
        body,
        grid=(NR, H),
        in_specs=[
            pl.BlockSpec((Hp, N, Fp), lambda i, h: (0, 0, 0)),
            pl.BlockSpec((H, Hp, Fp, Fo), lambda i, h: (0, 0, 0, 0)),
            pl.BlockSpec((H, 1, Fo), lambda i, h: (0, 0, 0)),
            pl.BlockSpec((H, 1, Fo), lambda i, h: (0, 0, 0)),
            pl.BlockSpec((BR, N), lambda i, h: (i, 0)),
        ],
        out_specs=pl.BlockSpec((1, BR, Fo), lambda i, h: (h, i, 0)),
        out_shape=jax.ShapeDtypeStruct((H, N, Fo), jnp.float32),
        scratch_shapes=[
            pltpu.VMEM((H, N, Fo + 1), jnp.float32),
            pltpu.VMEM((H, N, 1), jnp.float32),
            pltpu.VMEM((H, 1, N), jnp.float32),
            pltpu.VMEM((H, N, 1), jnp.float32),
        ],
    )(xh, Wr, attL, attR, maskadd)


def _readout(hi, hn, wdi, wdn, bd):
    def body(hi_ref, hn_ref, wdi_ref, wdn_ref, bd_ref, o_ref):
        si = jnp.sum(hi_ref[...], axis=1)
        sn = jnp.sum(hn_ref[...], axis=1)
        def tot(x):
            return jnp.sum(x, axis=(0, 1), keepdims=True)
        n2 = tot(si * si) + tot(sn * sn)
        nrm = jnp.maximum(jnp.sqrt(n2), jnp.float32(1e-12))
        y = (tot(si * wdi_ref[...]) + tot(sn * wdn_ref[...])) / nrm
        o_ref[...] = y + bd_ref[...]

    return pl.pallas_call(
        body,
        out_shape=jax.ShapeDtypeStruct((1, 1), jnp.float32),
    )(hi, hn, wdi, wdn, bd)


def kernel(v, a_int, a_nh, W1i, A1i, W1n, A1n, W2i, A2i, W2n, A2n,
           W3i, A3i, W3n, A3n, Wd, bd):
    m_int = _mask_prep(a_int)
    m_nh = _mask_prep(a_nh)

    def split_att(A, Fo):
        log2e = jnp.float32(1.4426950408889634)
        return A[:, None, :Fo] * log2e, A[:, None, Fo:] * log2e

    x0 = v.reshape(1, N, 11)
    a1l_i, a1r_i = split_att(A1i, 16)
    a1l_n, a1r_n = split_att(A1n, 16)
    hi = _gat(x0, m_int, W1i.reshape(H, 1, 11, 16), a1l_i, a1r_i)
    hn = _gat(x0, m_nh, W1n.reshape(H, 1, 11, 16), a1l_n, a1r_n)

    a2l_i, a2r_i = split_att(A2i, 32)
    a2l_n, a2r_n = split_att(A2n, 32)
    hi = _gat(hi, m_int, W2i.reshape(H, H, 16, 32), a2l_i, a2r_i)
    hn = _gat(hn, m_nh, W2n.reshape(H, H, 16, 32), a2l_n, a2r_n)

    a3l_i, a3r_i = split_att(A3i, 64)
    a3l_n, a3r_n = split_att(A3n, 64)
    hi = _gat(hi, m_int, W3i.reshape(H, H, 32, 64), a3l_i, a3r_i)
    hn = _gat(hn, m_nh, W3n.reshape(H, H, 32, 64), a3l_n, a3r_n)

    wdi = Wd[:H * 64, 0].reshape(H, 64)
    wdn = Wd[H * 64:, 0].reshape(H, 64)
    out = _readout(hi, hn, wdi, wdn, bd.reshape(1, 1))
    return out.reshape(1)


# speedup vs baseline: 3.0403x; 1.6464x over previous
import jax
import jax.numpy as jnp
from jax.experimental import pallas as pl
from jax.experimental.pallas import tpu as pltpu

N = 2048
H = 6
BR = 256
NR = N // BR
MASK_OFF = -1080.0


def _mask_prep(a):
    def body(a_ref, m_ref):
        i = pl.program_id(0)
        rows = jax.lax.broadcasted_iota(jnp.int32, (BR, N), 0) + i * BR
        cols = jax.lax.broadcasted_iota(jnp.int32, (BR, N), 1)
        m = (a_ref[...] > 0.99) | (rows == cols)
        m_ref[...] = jnp.where(m, 0.0, MASK_OFF).astype(jnp.bfloat16)

    return pl.pallas_call(
        body,
        grid=(NR,),
        in_specs=[pl.BlockSpec((BR, N), lambda i: (i, 0))],
        out_specs=pl.BlockSpec((BR, N), lambda i: (i, 0)),
        out_shape=jax.ShapeDtypeStruct((N, N), jnp.bfloat16),
    )(a)


def _gat(x, maskadd, W, ALR, AR):
    Fin = x.shape[1]
    Fo = W.shape[-1]

    def body(x_ref, w_ref, alr_ref, ar_ref, m_ref, o_ref,
             h_scr, b1_scr, b2_scr, fr_scr, fr2_scr):
        i = pl.program_id(0)

        @pl.when(i == 0)
        def _project():
            for g in range(H):
                acc = jnp.dot(x_ref[...], w_ref[g],
                              preferred_element_type=jnp.float32)
                h_scr[g, :, :Fo] = acc
                h_scr[g, :, Fo:] = jnp.ones((N, 1), jnp.float32)
                fld = jnp.dot(acc, alr_ref[g],
                              preferred_element_type=jnp.float32)
                frr = jax.lax.dot_general(
                    ar_ref[g], acc, (((1,), (1,)), ((), ())),
                    preferred_element_type=jnp.float32)
                fl = fld[:, 0:1]
                fd = fl + fld[:, 1:2]
                d = jnp.maximum(fd, 0.2 * fd)
                b1_scr[g] = fl - d
                b2_scr[g] = 0.2 * fl - d
                fr_scr[g] = frr
                fr2_scr[g] = 0.2 * frr

        for g in range(H):
            b1 = b1_scr[g, pl.ds(i * BR, BR), :]
            b2 = b2_scr[g, pl.ds(i * BR, BR), :]
            t = jnp.maximum(fr_scr[g] + b1, fr2_scr[g] + b2)
            p = jnp.exp2(t + m_ref[...].astype(jnp.float32))
            oext = jnp.dot(p, h_scr[g], preferred_element_type=jnp.float32)
            o = oext[:, :Fo] / oext[:, Fo:]
            o_ref[:, g * Fo:(g + 1) * Fo] = jnp.maximum(o, 0.0)

    return pl.pallas_call(
        body,
        grid=(NR,),
        in_specs=[
            pl.BlockSpec((N, Fin), lambda i: (0, 0)),
            pl.BlockSpec((H, Fin, Fo), lambda i: (0, 0, 0)),
            pl.BlockSpec((H, Fo, 2), lambda i: (0, 0, 0)),
            pl.BlockSpec((H, 1, Fo), lambda i: (0, 0, 0)),
            pl.BlockSpec((BR, N), lambda i: (i, 0)),
        ],
        out_specs=pl.BlockSpec((BR, H * Fo), lambda i: (i, 0)),
        out_shape=jax.ShapeDtypeStruct((N, H * Fo), jnp.float32),
        scratch_shapes=[
            pltpu.VMEM((H, N, Fo + 1), jnp.float32),
            pltpu.VMEM((H, N, 1), jnp.float32),
            pltpu.VMEM((H, N, 1), jnp.float32),
            pltpu.VMEM((H, 1, N), jnp.float32),
            pltpu.VMEM((H, 1, N), jnp.float32),
        ],
    )(x, W, ALR, AR, maskadd)


def _readout(hi, hn, wdi, wdn, bd):
    def body(hi_ref, hn_ref, wdi_ref, wdn_ref, bd_ref, o_ref):
        si = jnp.sum(hi_ref[...], axis=0, keepdims=True)
        sn = jnp.sum(hn_ref[...], axis=0, keepdims=True)
        def tot(x):
            return jnp.sum(x, axis=(0, 1), keepdims=True)
        n2 = tot(si * si) + tot(sn * sn)
        nrm = jnp.maximum(jnp.sqrt(n2), jnp.float32(1e-12))
        y = (tot(si * wdi_ref[...]) + tot(sn * wdn_ref[...])) / nrm
        o_ref[...] = y + bd_ref[...]

    return pl.pallas_call(
        body,
        out_shape=jax.ShapeDtypeStruct((1, 1), jnp.float32),
    )(hi, hn, wdi, wdn, bd)


def _att_prep(A, Fo):
    log2e = jnp.float32(1.4426950408889634)
    alr = jnp.stack([A[:, :Fo], A[:, Fo:]], axis=-1) * log2e
    ar = A[:, None, Fo:] * log2e
    return alr, ar


def kernel(v, a_int, a_nh, W1i, A1i, W1n, A1n, W2i, A2i, W2n, A2n,
           W3i, A3i, W3n, A3n, Wd, bd):
    m_int = _mask_prep(a_int)
    m_nh = _mask_prep(a_nh)

    alr1i, ar1i = _att_prep(A1i, 16)
    alr1n, ar1n = _att_prep(A1n, 16)
    hi = _gat(v, m_int, W1i, alr1i, ar1i)
    hn = _gat(v, m_nh, W1n, alr1n, ar1n)

    alr2i, ar2i = _att_prep(A2i, 32)
    alr2n, ar2n = _att_prep(A2n, 32)
    hi = _gat(hi, m_int, W2i, alr2i, ar2i)
    hn = _gat(hn, m_nh, W2n, alr2n, ar2n)

    alr3i, ar3i = _att_prep(A3i, 64)
    alr3n, ar3n = _att_prep(A3n, 64)
    hi = _gat(hi, m_int, W3i, alr3i, ar3i)
    hn = _gat(hn, m_nh, W3n, alr3n, ar3n)

    wdi = Wd[:H * 64, 0].reshape(1, H * 64)
    wdn = Wd[H * 64:, 0].reshape(1, H * 64)
    out = _readout(hi, hn, wdi, wdn, bd.reshape(1, 1))
    return out.reshape(1)


# speedup vs baseline: 3.4717x; 1.1419x over previous
import jax
import jax.numpy as jnp
from jax.experimental import pallas as pl
from jax.experimental.pallas import tpu as pltpu

N = 2048
H = 6
BR = 256
NR = N // BR
MASK_OFF = -1080.0


def _mask_prep(a_int, a_nh):
    def body(ai_ref, an_ref, m_ref):
        b = pl.program_id(0)
        i = pl.program_id(1)
        rows = jax.lax.broadcasted_iota(jnp.int32, (BR, N), 0) + i * BR
        cols = jax.lax.broadcasted_iota(jnp.int32, (BR, N), 1)
        sel = jnp.where(b == 0, ai_ref[...], an_ref[...])
        m = (sel > 0.99) | (rows == cols)
        m_ref[0] = jnp.where(m, 0.0, MASK_OFF).astype(jnp.bfloat16)

    return pl.pallas_call(
        body,
        grid=(2, NR),
        in_specs=[
            pl.BlockSpec((BR, N), lambda b, i: (i * (1 - b), 0)),
            pl.BlockSpec((BR, N), lambda b, i: (i * b, 0)),
        ],
        out_specs=pl.BlockSpec((1, BR, N), lambda b, i: (b, i, 0)),
        out_shape=jax.ShapeDtypeStruct((2, N, N), jnp.bfloat16),
    )(a_int, a_nh)


def _gat(x, masks, W, ALR, AR):
    Bx, _, Fin = x.shape
    Fo = W.shape[-1]

    def body(x_ref, w_ref, alr_ref, ar_ref, m_ref, o_ref,
             h_scr, b1_scr, b2_scr, fr_scr, fr2_scr):
        i = pl.program_id(1)

        @pl.when(i == 0)
        def _project():
            for g in range(H):
                acc = jnp.dot(x_ref[0], w_ref[0, g],
                              preferred_element_type=jnp.float32)
                h_scr[g, :, :Fo] = acc
                h_scr[g, :, Fo:] = jnp.ones((N, 1), jnp.float32)
                fld = jnp.dot(acc, alr_ref[0, g],
                              preferred_element_type=jnp.float32)
                frr = jax.lax.dot_general(
                    ar_ref[0, g], acc, (((1,), (1,)), ((), ())),
                    preferred_element_type=jnp.float32)
                fl = fld[:, 0:1]
                fd = fl + fld[:, 1:2]
                d = jnp.maximum(fd, 0.2 * fd)
                b1_scr[g] = fl - d
                b2_scr[g] = 0.2 * fl - d
                fr_scr[g] = frr
                fr2_scr[g] = 0.2 * frr

        madd = m_ref[0].astype(jnp.float32)
        for g in range(H):
            b1 = b1_scr[g, pl.ds(i * BR, BR), :]
            b2 = b2_scr[g, pl.ds(i * BR, BR), :]
            t = jnp.maximum(fr_scr[g] + b1, fr2_scr[g] + b2)
            p = jnp.exp2(t + madd)
            oext = jnp.dot(p, h_scr[g], preferred_element_type=jnp.float32)
            o = oext[:, :Fo] / oext[:, Fo:]
            o_ref[0, :, g * Fo:(g + 1) * Fo] = jnp.maximum(o, 0.0)

    xmap = (lambda b, i: (b, 0, 0)) if Bx == 2 else (lambda b, i: (0, 0, 0))
    return pl.pallas_call(
        body,
        grid=(2, NR),
        in_specs=[
            pl.BlockSpec((1, N, Fin), xmap),
            pl.BlockSpec((1, H, Fin, Fo), lambda b, i: (b, 0, 0, 0)),
            pl.BlockSpec((1, H, Fo, 2), lambda b, i: (b, 0, 0, 0)),
            pl.BlockSpec((1, H, 1, Fo), lambda b, i: (b, 0, 0, 0)),
            pl.BlockSpec((1, BR, N), lambda b, i: (b, i, 0)),
        ],
        out_specs=pl.BlockSpec((1, BR, H * Fo), lambda b, i: (b, i, 0)),
        out_shape=jax.ShapeDtypeStruct((2, N, H * Fo), jnp.float32),
        scratch_shapes=[
            pltpu.VMEM((H, N, Fo + 1), jnp.float32),
            pltpu.VMEM((H, N, 1), jnp.float32),
            pltpu.VMEM((H, N, 1), jnp.float32),
            pltpu.VMEM((H, 1, N), jnp.float32),
            pltpu.VMEM((H, 1, N), jnp.float32),
        ],
    )(x, W, ALR, AR, masks)


def _readout(h, wd, bd):
    def body(h_ref, wd_ref, bd_ref, o_ref):
        s = jnp.sum(h_ref[...], axis=1)
        def tot(x):
            return jnp.sum(x, axis=(0, 1), keepdims=True)
        n2 = tot(s * s)
        nrm = jnp.maximum(jnp.sqrt(n2), jnp.float32(1e-12))
        y = tot(s * wd_ref[...]) / nrm
        o_ref[...] = y + bd_ref[...]

    return pl.pallas_call(
        body,
        out_shape=jax.ShapeDtypeStruct((1, 1), jnp.float32),
    )(h, wd, bd)


def _att_prep(Ai, An, Fo):
    log2e = jnp.float32(1.4426950408889634)
    A2 = jnp.stack([Ai, An]) * log2e
    alr = jnp.stack([A2[:, :, :Fo], A2[:, :, Fo:]], axis=-1)
    ar = A2[:, :, None, Fo:]
    return alr, ar


def kernel(v, a_int, a_nh, W1i, A1i, W1n, A1n, W2i, A2i, W2n, A2n,
           W3i, A3i, W3n, A3n, Wd, bd):
    masks = _mask_prep(a_int, a_nh)

    alr1, ar1 = _att_prep(A1i, A1n, 16)
    h = _gat(v[None], masks, jnp.stack([W1i, W1n]), alr1, ar1)
    alr2, ar2 = _att_prep(A2i, A2n, 32)
    h = _gat(h, masks, jnp.stack([W2i, W2n]), alr2, ar2)
    alr3, ar3 = _att_prep(A3i, A3n, 64)
    h = _gat(h, masks, jnp.stack([W3i, W3n]), alr3, ar3)

    out = _readout(h, Wd[:, 0].reshape(2, H * 64), bd.reshape(1, 1))
    return out.reshape(1)


# speedup vs baseline: 3.6487x; 1.0510x over previous
import jax
import jax.numpy as jnp
from jax.experimental import pallas as pl
from jax.experimental.pallas import tpu as pltpu

N = 2048
H = 6
BR = 512
NR = N // BR
MASK_OFF = -1080.0


def _mask_prep(a_int, a_nh):
    def body(ai_ref, an_ref, m_ref):
        b = pl.program_id(0)
        i = pl.program_id(1)
        rows = jax.lax.broadcasted_iota(jnp.int32, (BR, N), 0) + i * BR
        cols = jax.lax.broadcasted_iota(jnp.int32, (BR, N), 1)
        sel = jnp.where(b == 0, ai_ref[...], an_ref[...])
        m = (sel > 0.99) | (rows == cols)
        m_ref[0] = jnp.where(m, 0.0, MASK_OFF).astype(jnp.bfloat16)

    return pl.pallas_call(
        body,
        grid=(2, NR),
        in_specs=[
            pl.BlockSpec((BR, N), lambda b, i: (i * (1 - b), 0)),
            pl.BlockSpec((BR, N), lambda b, i: (i * b, 0)),
        ],
        out_specs=pl.BlockSpec((1, BR, N), lambda b, i: (b, i, 0)),
        out_shape=jax.ShapeDtypeStruct((2, N, N), jnp.bfloat16),
    )(a_int, a_nh)


def _gat(x, masks, W, ALR, AR):
    Bx, _, Fin = x.shape
    Fo = W.shape[-1]

    def body(x_ref, w_ref, alr_ref, ar_ref, m_ref, o_ref,
             h_scr, b1_scr, b2_scr, fr_scr, fr2_scr):
        bb = pl.program_id(0)
        i = pl.program_id(1)

        @pl.when((bb == 0) & (i == 0))
        def _ones():
            for g in range(H):
                h_scr[g, :, Fo:] = jnp.ones((N, 1), jnp.float32)

        @pl.when(i == 0)
        def _project():
            for g in range(H):
                acc = jnp.dot(x_ref[0], w_ref[0, g],
                              preferred_element_type=jnp.float32)
                h_scr[g, :, :Fo] = acc
                fld = jnp.dot(acc, alr_ref[0, g],
                              preferred_element_type=jnp.float32)
                frr = jax.lax.dot_general(
                    ar_ref[0, g], acc, (((1,), (1,)), ((), ())),
                    preferred_element_type=jnp.float32)
                fl = fld[:, 0:1]
                fd = fl + fld[:, 1:2]
                d = jnp.maximum(fd, 0.2 * fd)
                b1_scr[g] = fl - d
                b2_scr[g] = 0.2 * fl - d
                fr_scr[g] = frr
                fr2_scr[g] = 0.2 * frr

        madd = m_ref[0].astype(jnp.float32)
        for g in range(H):
            b1 = b1_scr[g, pl.ds(i * BR, BR), :]
            b2 = b2_scr[g, pl.ds(i * BR, BR), :]
            t = jnp.maximum(fr_scr[g] + b1, fr2_scr[g] + b2)
            p = jnp.exp2(t + madd)
            oext = jnp.dot(p, h_scr[g], preferred_element_type=jnp.float32)
            o = oext[:, :Fo] / oext[:, Fo:]
            o_ref[0, :, g * Fo:(g + 1) * Fo] = jnp.maximum(o, 0.0)

    xmap = (lambda b, i: (b, 0, 0)) if Bx == 2 else (lambda b, i: (0, 0, 0))
    return pl.pallas_call(
        body,
        grid=(2, NR),
        in_specs=[
            pl.BlockSpec((1, N, Fin), xmap),
            pl.BlockSpec((1, H, Fin, Fo), lambda b, i: (b, 0, 0, 0)),
            pl.BlockSpec((1, H, Fo, 2), lambda b, i: (b, 0, 0, 0)),
            pl.BlockSpec((1, H, 1, Fo), lambda b, i: (b, 0, 0, 0)),
            pl.BlockSpec((1, BR, N), lambda b, i: (b, i, 0)),
        ],
        out_specs=pl.BlockSpec((1, BR, H * Fo), lambda b, i: (b, i, 0)),
        out_shape=jax.ShapeDtypeStruct((2, N, H * Fo), jnp.float32),
        scratch_shapes=[
            pltpu.VMEM((H, N, Fo + 1), jnp.float32),
            pltpu.VMEM((H, N, 1), jnp.float32),
            pltpu.VMEM((H, N, 1), jnp.float32),
            pltpu.VMEM((H, 1, N), jnp.float32),
            pltpu.VMEM((H, 1, N), jnp.float32),
        ],
    )(x, W, ALR, AR, masks)


def _readout(h, wd, bd):
    def body(h_ref, wd_ref, bd_ref, o_ref):
        s = jnp.sum(h_ref[...], axis=1)
        def tot(x):
            return jnp.sum(x, axis=(0, 1), keepdims=True)
        n2 = tot(s * s)
        nrm = jnp.maximum(jnp.sqrt(n2), jnp.float32(1e-12))
        y = tot(s * wd_ref[...]) / nrm
        o_ref[...] = y + bd_ref[...]

    return pl.pallas_call(
        body,
        out_shape=jax.ShapeDtypeStruct((1, 1), jnp.float32),
    )(h, wd, bd)


def _att_prep(Ai, An, Fo):
    log2e = jnp.float32(1.4426950408889634)
    A2 = jnp.stack([Ai, An]) * log2e
    alr = jnp.stack([A2[:, :, :Fo], A2[:, :, Fo:]], axis=-1)
    ar = A2[:, :, None, Fo:]
    return alr, ar


def kernel(v, a_int, a_nh, W1i, A1i, W1n, A1n, W2i, A2i, W2n, A2n,
           W3i, A3i, W3n, A3n, Wd, bd):
    masks = _mask_prep(a_int, a_nh)

    alr1, ar1 = _att_prep(A1i, A1n, 16)
    h = _gat(v[None], masks, jnp.stack([W1i, W1n]), alr1, ar1)
    alr2, ar2 = _att_prep(A2i, A2n, 32)
    h = _gat(h, masks, jnp.stack([W2i, W2n]), alr2, ar2)
    alr3, ar3 = _att_prep(A3i, A3n, 64)
    h = _gat(h, masks, jnp.stack([W3i, W3n]), alr3, ar3)

    out = _readout(h, Wd[:, 0].reshape(2, H * 64), bd.reshape(1, 1))
    return out.reshape(1)


# speedup vs baseline: 3.7391x; 1.0248x over previous
import jax
import jax.numpy as jnp
from jax.experimental import pallas as pl
from jax.experimental.pallas import tpu as pltpu

N = 2048
H = 6
BR = 512
NR = N // BR
MASK_OFF = -1080.0


def _mask_prep(a_int, a_nh):
    def body(ai_ref, an_ref, m_ref):
        b = pl.program_id(0)
        i = pl.program_id(1)
        rows = jax.lax.broadcasted_iota(jnp.int32, (BR, N), 0) + i * BR
        cols = jax.lax.broadcasted_iota(jnp.int32, (BR, N), 1)
        sel = jnp.where(b == 0, ai_ref[...], an_ref[...])
        m = (sel > 0.99) | (rows == cols)
        m_ref[0] = jnp.where(m, 0.0, MASK_OFF).astype(jnp.bfloat16)

    return pl.pallas_call(
        body,
        grid=(2, NR),
        in_specs=[
            pl.BlockSpec((BR, N), lambda b, i: (i * (1 - b), 0)),
            pl.BlockSpec((BR, N), lambda b, i: (i * b, 0)),
        ],
        out_specs=pl.BlockSpec((1, BR, N), lambda b, i: (b, i, 0)),
        out_shape=jax.ShapeDtypeStruct((2, N, N), jnp.bfloat16),
    )(a_int, a_nh)


def _gat(x, masks, W, ALR, AR):
    Bx, _, Fin = x.shape
    Fo = W.shape[-1]

    def body(x_ref, w_ref, alr_ref, ar_ref, m_ref, o_ref,
             h_scr, b1_scr, b2_scr, fr_scr, fr2_scr):
        bb = pl.program_id(0)
        i = pl.program_id(1)

        @pl.when((bb == 0) & (i == 0))
        def _ones():
            for g in range(H):
                h_scr[g, :, Fo:] = jnp.ones((N, 1), jnp.float32)

        @pl.when(i == 0)
        def _project():
            for g in range(H):
                acc = jnp.dot(x_ref[0], w_ref[0, g],
                              preferred_element_type=jnp.float32)
                h_scr[g, :, :Fo] = acc
                fld = jnp.dot(acc, alr_ref[0, g],
                              preferred_element_type=jnp.float32)
                frr = jax.lax.dot_general(
                    ar_ref[0, g], acc, (((1,), (1,)), ((), ())),
                    preferred_element_type=jnp.float32)
                fl = fld[:, 0:1]
                fd = fl + fld[:, 1:2]
                d = jnp.maximum(fd, 0.2 * fd)
                b1_scr[g] = fl - d
                b2_scr[g] = 0.2 * fl - d
                fr_scr[g] = frr
                fr2_scr[g] = 0.2 * frr

        madd = m_ref[0].astype(jnp.float32)
        for g in range(H):
            b1 = b1_scr[g, pl.ds(i * BR, BR), :]
            b2 = b2_scr[g, pl.ds(i * BR, BR), :]
            t = jnp.maximum(fr_scr[g] + b1, fr2_scr[g] + b2)
            p = jnp.exp2(t + madd)
            oext = jnp.dot(p, h_scr[g], preferred_element_type=jnp.float32)
            o = oext[:, :Fo] / oext[:, Fo:]
            o_ref[0, :, g * Fo:(g + 1) * Fo] = jnp.maximum(o, 0.0)

    xmap = (lambda b, i: (b, 0, 0)) if Bx == 2 else (lambda b, i: (0, 0, 0))
    return pl.pallas_call(
        body,
        grid=(2, NR),
        in_specs=[
            pl.BlockSpec((1, N, Fin), xmap),
            pl.BlockSpec((1, H, Fin, Fo), lambda b, i: (b, 0, 0, 0)),
            pl.BlockSpec((1, H, Fo, 2), lambda b, i: (b, 0, 0, 0)),
            pl.BlockSpec((1, H, 1, Fo), lambda b, i: (b, 0, 0, 0)),
            pl.BlockSpec((1, BR, N), lambda b, i: (b, i, 0)),
        ],
        out_specs=pl.BlockSpec((1, BR, H * Fo), lambda b, i: (b, i, 0)),
        out_shape=jax.ShapeDtypeStruct((2, N, H * Fo), jnp.float32),
        scratch_shapes=[
            pltpu.VMEM((H, N, Fo + 1), jnp.float32),
            pltpu.VMEM((H, N, 1), jnp.float32),
            pltpu.VMEM((H, N, 1), jnp.float32),
            pltpu.VMEM((H, 1, N), jnp.float32),
            pltpu.VMEM((H, 1, N), jnp.float32),
        ],
    )(x, W, ALR, AR, masks)


def _gat_final(x, masks, W, ALR, AR, wd, bd):
    _, _, Fin = x.shape
    Fo = W.shape[-1]

    def body(x_ref, w_ref, alr_ref, ar_ref, m_ref, wd_ref, wd2_ref, bd_ref,
             o_ref, h_scr, b1_scr, b2_scr, fr_scr, fr2_scr, s_scr):
        bb = pl.program_id(0)
        i = pl.program_id(1)

        @pl.when((bb == 0) & (i == 0))
        def _ones():
            for g in range(H):
                h_scr[g, :, Fo:] = jnp.ones((N, 1), jnp.float32)
            s_scr[...] = jnp.zeros((2, 1, H * Fo), jnp.float32)

        @pl.when(i == 0)
        def _project():
            for g in range(H):
                acc = jnp.dot(x_ref[0], w_ref[0, g],
                              preferred_element_type=jnp.float32)
                h_scr[g, :, :Fo] = acc
                fld = jnp.dot(acc, alr_ref[0, g],
                              preferred_element_type=jnp.float32)
                frr = jax.lax.dot_general(
                    ar_ref[0, g], acc, (((1,), (1,)), ((), ())),
                    preferred_element_type=jnp.float32)
                fl = fld[:, 0:1]
                fd = fl + fld[:, 1:2]
                d = jnp.maximum(fd, 0.2 * fd)
                b1_scr[g] = fl - d
                b2_scr[g] = 0.2 * fl - d
                fr_scr[g] = frr
                fr2_scr[g] = 0.2 * frr

        madd = m_ref[0].astype(jnp.float32)
        cols = []
        for g in range(H):
            b1 = b1_scr[g, pl.ds(i * BR, BR), :]
            b2 = b2_scr[g, pl.ds(i * BR, BR), :]
            t = jnp.maximum(fr_scr[g] + b1, fr2_scr[g] + b2)
            p = jnp.exp2(t + madd)
            oext = jnp.dot(p, h_scr[g], preferred_element_type=jnp.float32)
            o = jnp.maximum(oext[:, :Fo] / oext[:, Fo:], 0.0)
            cols.append(jnp.sum(o, axis=0, keepdims=True))
        part = jnp.concatenate(cols, axis=1)

        @pl.when(bb == 0)
        def _acc0():
            s_scr[0] = s_scr[0] + part

        @pl.when(bb == 1)
        def _acc1():
            s_scr[1] = s_scr[1] + part

        @pl.when((bb == 1) & (i == NR - 1))
        def _flush():
            def tot(z):
                return jnp.sum(z, axis=(0, 1), keepdims=True)
            s0 = s_scr[0]
            s1 = s_scr[1]
            n2 = tot(s0 * s0) + tot(s1 * s1)
            num = tot(s0 * wd_ref[0]) + tot(s1 * wd2_ref[0])
            nrm = jnp.maximum(jnp.sqrt(n2), jnp.float32(1e-12))
            o_ref[...] = num / nrm + bd_ref[...]

    return pl.pallas_call(
        body,
        grid=(2, NR),
        in_specs=[
            pl.BlockSpec((1, N, Fin), lambda b, i: (b, 0, 0)),
            pl.BlockSpec((1, H, Fin, Fo), lambda b, i: (b, 0, 0, 0)),
            pl.BlockSpec((1, H, Fo, 2), lambda b, i: (b, 0, 0, 0)),
            pl.BlockSpec((1, H, 1, Fo), lambda b, i: (b, 0, 0, 0)),
            pl.BlockSpec((1, BR, N), lambda b, i: (b, i, 0)),
            pl.BlockSpec((1, 1, H * Fo), lambda b, i: (0, 0, 0)),
            pl.BlockSpec((1, 1, H * Fo), lambda b, i: (1, 0, 0)),
            pl.BlockSpec((1, 1), lambda b, i: (0, 0)),
        ],
        out_specs=pl.BlockSpec((1, 1), lambda b, i: (0, 0)),
        out_shape=jax.ShapeDtypeStruct((1, 1), jnp.float32),
        scratch_shapes=[
            pltpu.VMEM((H, N, Fo + 1), jnp.float32),
            pltpu.VMEM((H, N, 1), jnp.float32),
            pltpu.VMEM((H, N, 1), jnp.float32),
            pltpu.VMEM((H, 1, N), jnp.float32),
            pltpu.VMEM((H, 1, N), jnp.float32),
            pltpu.VMEM((2, 1, H * Fo), jnp.float32),
        ],
    )(x, W, ALR, AR, masks, wd, wd, bd)


def _att_prep(Ai, An, Fo):
    log2e = jnp.float32(1.4426950408889634)
    A2 = jnp.stack([Ai, An]) * log2e
    alr = jnp.stack([A2[:, :, :Fo], A2[:, :, Fo:]], axis=-1)
    ar = A2[:, :, None, Fo:]
    return alr, ar


def kernel(v, a_int, a_nh, W1i, A1i, W1n, A1n, W2i, A2i, W2n, A2n,
           W3i, A3i, W3n, A3n, Wd, bd):
    masks = _mask_prep(a_int, a_nh)

    alr1, ar1 = _att_prep(A1i, A1n, 16)
    h = _gat(v[None], masks, jnp.stack([W1i, W1n]), alr1, ar1)
    alr2, ar2 = _att_prep(A2i, A2n, 32)
    h = _gat(h, masks, jnp.stack([W2i, W2n]), alr2, ar2)
    alr3, ar3 = _att_prep(A3i, A3n, 64)
    out = _gat_final(h, masks, jnp.stack([W3i, W3n]), alr3, ar3,
                     Wd[:, 0].reshape(2, 1, H * 64), bd.reshape(1, 1))
    return out.reshape(1)
